# baseline (speedup 1.0000x reference)
import functools

import jax
import jax.numpy as jnp
from jax import lax
from jax.experimental import pallas as pl
from jax.experimental.pallas import tpu as pltpu
from jax.experimental.pallas import tpu_sc as plsc

HIDDEN = 1024
EXPERT_INTER = 1024
NUM_EXPERTS = 16
VOCAB = 100000
N_TOKENS = 128


def _route_body(tid_hbm, eid_hbm, tid_v, eid_v):
    c = lax.axis_index("c")
    s = lax.axis_index("s")

    @pl.when((c == 0) & (s == 0))
    def _():
        pltpu.sync_copy(tid_hbm, tid_v)
        for i in range(N_TOKENS // 16):
            t = tid_v[pl.ds(i * 16, 16)]
            t = jnp.clip(t, 0, VOCAB - 1)
            eid_v[pl.ds(i * 16, 16)] = jax.lax.rem(t, NUM_EXPERTS)
        pltpu.sync_copy(eid_v, eid_hbm)


def _route_sc(token_ids):
    mesh = plsc.VectorSubcoreMesh(core_axis_name="c", subcore_axis_name="s")
    fn = functools.partial(
        pl.kernel,
        mesh=mesh,
        out_type=jax.ShapeDtypeStruct((N_TOKENS,), jnp.int32),
        scratch_types=[
            pltpu.VMEM((N_TOKENS,), jnp.int32),
            pltpu.VMEM((N_TOKENS,), jnp.int32),
        ],
    )(_route_body)
    return fn(token_ids)


def _moe_body(eid_ref, x_ref, gatew_ref, upw_ref, dnw_ref, out_ref):
    e = pl.program_id(0)

    @pl.when(e == 0)
    def _init():
        out_ref[...] = jnp.zeros_like(out_ref)

    x = x_ref[...].astype(jnp.bfloat16)
    gate = jnp.dot(x, gatew_ref[0].astype(jnp.bfloat16),
                   preferred_element_type=jnp.float32)
    up = jnp.dot(x, upw_ref[0].astype(jnp.bfloat16),
                 preferred_element_type=jnp.float32)
    act = gate * jax.nn.sigmoid(gate) * up
    y = jnp.dot(act.astype(jnp.bfloat16), dnw_ref[0].astype(jnp.bfloat16),
                preferred_element_type=jnp.float32)

    mask = eid_ref[...] == e
    out_ref[...] += jnp.where(mask, y, 0.0)


def kernel(x, token_ids, gate_up_proj, down_proj):
    n = x.shape[0]
    eid = _route_sc(token_ids.astype(jnp.int32)).reshape(n, 1)
    return pl.pallas_call(
        _moe_body,
        grid=(NUM_EXPERTS,),
        in_specs=[
            pl.BlockSpec((n, 1), lambda e: (0, 0)),
            pl.BlockSpec((n, HIDDEN), lambda e: (0, 0)),
            pl.BlockSpec((1, HIDDEN, EXPERT_INTER), lambda e: (e, 0, 0)),
            pl.BlockSpec((1, HIDDEN, EXPERT_INTER), lambda e: (e, 0, 1)),
            pl.BlockSpec((1, EXPERT_INTER, HIDDEN), lambda e: (e, 0, 0)),
        ],
        out_specs=pl.BlockSpec((n, HIDDEN), lambda e: (0, 0)),
        out_shape=jax.ShapeDtypeStruct((n, HIDDEN), jnp.float32),
        compiler_params=pltpu.CompilerParams(
            dimension_semantics=("arbitrary",),
        ),
    )(eid, x, gate_up_proj, gate_up_proj, down_proj)


# speedup vs baseline: 1.3245x; 1.3245x over previous
import jax
import jax.numpy as jnp
from jax.experimental import pallas as pl
from jax.experimental.pallas import tpu as pltpu

HIDDEN = 1024
EXPERT_INTER = 1024
NUM_EXPERTS = 16
VOCAB = 100000
N_TOKENS = 128


def _moe_body(tid_ref, x_ref, gatew_ref, upw_ref, dnw_ref, out_ref):
    e = pl.program_id(0)

    @pl.when(e == 0)
    def _init():
        out_ref[...] = jnp.zeros_like(out_ref)

    tid = jnp.clip(tid_ref[...], 0, VOCAB - 1)
    eid = jax.lax.rem(tid, NUM_EXPERTS)
    mask = eid == e
    x = jnp.where(mask, x_ref[...], 0.0).astype(jnp.bfloat16)
    gate = jnp.dot(x, gatew_ref[0].astype(jnp.bfloat16),
                   preferred_element_type=jnp.float32)
    up = jnp.dot(x, upw_ref[0].astype(jnp.bfloat16),
                 preferred_element_type=jnp.float32)
    act = gate * jax.nn.sigmoid(gate) * up
    y = jnp.dot(act.astype(jnp.bfloat16), dnw_ref[0].astype(jnp.bfloat16),
                preferred_element_type=jnp.float32)
    out_ref[...] += y


def kernel(x, token_ids, gate_up_proj, down_proj):
    n = x.shape[0]
    tid2d = token_ids.reshape(n, 1).astype(jnp.int32)
    return pl.pallas_call(
        _moe_body,
        grid=(NUM_EXPERTS,),
        in_specs=[
            pl.BlockSpec((n, 1), lambda e: (0, 0)),
            pl.BlockSpec((n, HIDDEN), lambda e: (0, 0)),
            pl.BlockSpec((1, HIDDEN, EXPERT_INTER), lambda e: (e, 0, 0)),
            pl.BlockSpec((1, HIDDEN, EXPERT_INTER), lambda e: (e, 0, 1)),
            pl.BlockSpec((1, EXPERT_INTER, HIDDEN), lambda e: (e, 0, 0)),
        ],
        out_specs=pl.BlockSpec((n, HIDDEN), lambda e: (0, 0)),
        out_shape=jax.ShapeDtypeStruct((n, HIDDEN), jnp.float32),
        compiler_params=pltpu.CompilerParams(
            dimension_semantics=("arbitrary",),
        ),
    )(tid2d, x, gate_up_proj, gate_up_proj, down_proj)
